# baseline (speedup 1.0000x reference)
import jax
import jax.numpy as jnp
from jax import lax
from jax.experimental import pallas as pl
from jax.experimental.pallas import tpu as pltpu
from jax.experimental.pallas import tpu_sc as plsc

D_MODEL = 128


def _pe_lookup_body(beta_hbm, pe_hbm, dims_hbm, out_hbm, beta_s, dims_s, sem):
    cp1 = pltpu.async_copy(beta_hbm, beta_s, sem)
    cp2 = pltpu.async_copy(dims_hbm, dims_s, sem)
    cp1.wait()
    cp2.wait()
    idx = (
        beta_s[0] * dims_s[0]
        + beta_s[1] * dims_s[1]
        + beta_s[2] * dims_s[2]
    )
    pltpu.sync_copy(pe_hbm.at[pl.ds(idx, 1)], out_hbm)


def kernel(beta, pe, beta_dims):
    max_len = pe.shape[0]
    table = pe.reshape(max_len, D_MODEL)

    mesh = plsc.ScalarSubcoreMesh(axis_name="c", num_cores=1)
    out = pl.kernel(
        _pe_lookup_body,
        out_type=jax.ShapeDtypeStruct((1, D_MODEL), jnp.float32),
        mesh=mesh,
        scratch_types=[
            pltpu.SMEM((3,), jnp.int32),
            pltpu.SMEM((3,), jnp.int32),
            pltpu.SemaphoreType.DMA,
        ],
        compiler_params=pltpu.CompilerParams(
            skip_device_barrier=True,
            disable_bounds_checks=True,
            disable_semaphore_checks=True,
        ),
    )(beta, table, beta_dims)
    return out
